# baseline (speedup 1.0000x reference)
import functools

import jax
import jax.numpy as jnp
from jax.experimental import pallas as pl
from jax.experimental.pallas import tpu as pltpu

_K = 16
_J = 5
_BIG = 2**30
_BLOCK_R = 512
_W = 64


def _knn_block_kernel(pj_ref, pcols_ref, out_ref, *, n, k):
    xi = pcols_ref[0:1, :]
    yi = pcols_ref[1:2, :]
    zi = pcols_ref[2:3, :]
    csz = n // _W

    def dist_slice(w):
        dx = pj_ref[pl.ds(w * csz, csz), 0:1] - xi
        dy = pj_ref[pl.ds(w * csz, csz), 1:2] - yi
        dz = pj_ref[pl.ds(w * csz, csz), 2:3] - zi
        return dx * dx + dy * dy + dz * dz

    t = [dist_slice(w) for w in range(_W)]

    def tournament(slices):
        cur = []
        for a in range(0, _W, 2):
            c = slices[a + 1] < slices[a]
            cur.append((jnp.where(c, slices[a + 1], slices[a]),
                        jnp.where(c, a + 1, a)))
        while len(cur) > 1:
            nxt = []
            for a in range(0, len(cur), 2):
                v0, w0 = cur[a]
                v1, w1 = cur[a + 1]
                c = v1 < v0
                nxt.append((jnp.where(c, v1, v0), jnp.where(c, w1, w0)))
            cur = nxt
        return cur[0]

    si = jax.lax.broadcasted_iota(jnp.int32, (csz, _BLOCK_R), 0)
    vals, gidx = [], []
    for _ in range(_J):
        m, wdep = tournament(t)
        vals.append(m)
        gidx.append(wdep * csz + si)
        t = [jnp.where(wdep == w, jnp.inf, t[w]) for w in range(_W)]
    cand_v = jnp.concatenate(vals, axis=0)
    cand_i = jnp.concatenate(gidx, axis=0)

    rows = []
    m = None
    for _ in range(k):
        m = jnp.min(cand_v, axis=0, keepdims=True)
        idx = jnp.min(jnp.where(cand_v == m, cand_i, _BIG), axis=0,
                      keepdims=True)
        rows.append(idx)
        cand_v = jnp.where(cand_i == idx, jnp.inf, cand_v)
    out_ref[:, :] = jnp.concatenate(rows, axis=0)

    bad = jnp.any(vals[-1] <= m)

    @pl.when(bad)
    def _fallback():
        tt = jnp.concatenate([dist_slice(w) for w in range(_W)], axis=0)
        iota = jax.lax.broadcasted_iota(jnp.int32, tt.shape, 0)
        frows = []
        for _ in range(k):
            fm = jnp.min(tt, axis=0, keepdims=True)
            fidx = jnp.min(jnp.where(tt == fm, iota, _BIG), axis=0,
                           keepdims=True)
            frows.append(fidx)
            tt = jnp.where(iota == fidx, jnp.inf, tt)
        out_ref[:, :] = jnp.concatenate(frows, axis=0)


@jax.jit
def kernel(p, active_nodes):
    n = p.shape[0]
    pcols = jnp.zeros((8, n), dtype=p.dtype).at[:3, :].set(p.T)
    idxs_t = pl.pallas_call(
        functools.partial(_knn_block_kernel, n=n, k=_K),
        grid=(n // _BLOCK_R,),
        in_specs=[
            pl.BlockSpec((n, 3), lambda i: (0, 0)),
            pl.BlockSpec((8, _BLOCK_R), lambda i: (0, i)),
        ],
        out_specs=pl.BlockSpec((_K, _BLOCK_R), lambda i: (0, i)),
        out_shape=jax.ShapeDtypeStruct((_K, n), jnp.int32),
        compiler_params=pltpu.CompilerParams(
            dimension_semantics=("arbitrary",),
        ),
    )(p, pcols)
    idxs = idxs_t.T
    row = jnp.broadcast_to(jnp.arange(n, dtype=idxs.dtype)[:, None], (n, _K))
    s = jnp.where(active_nodes[:, None], idxs, n - 1)
    r = jnp.where(active_nodes[:, None], row, n - 1)
    return s.reshape(-1), r.reshape(-1)


# speedup vs baseline: 1.3967x; 1.3967x over previous
import functools

import jax
import jax.numpy as jnp
from jax.experimental import pallas as pl
from jax.experimental.pallas import tpu as pltpu

_K = 16
_J = 5
_BIG = 2**30
_BLOCK_R = 256
_W = 64


def _knn_block_kernel(pj_ref, pcols_ref, out_ref, *, n, k):
    xi = pcols_ref[0:1, :]
    yi = pcols_ref[1:2, :]
    zi = pcols_ref[2:3, :]
    csz = n // _W

    def dist_slice(w):
        dx = pj_ref[pl.ds(w * csz, csz), 0:1] - xi
        dy = pj_ref[pl.ds(w * csz, csz), 1:2] - yi
        dz = pj_ref[pl.ds(w * csz, csz), 2:3] - zi
        return dx * dx + dy * dy + dz * dz

    t = [dist_slice(w) for w in range(_W)]

    def tournament(slices):
        cur = []
        for a in range(0, _W, 2):
            c = slices[a + 1] < slices[a]
            cur.append((jnp.where(c, slices[a + 1], slices[a]),
                        jnp.where(c, a + 1, a)))
        while len(cur) > 1:
            nxt = []
            for a in range(0, len(cur), 2):
                v0, w0 = cur[a]
                v1, w1 = cur[a + 1]
                c = v1 < v0
                nxt.append((jnp.where(c, v1, v0), jnp.where(c, w1, w0)))
            cur = nxt
        return cur[0]

    si = jax.lax.broadcasted_iota(jnp.int32, (csz, _BLOCK_R), 0)
    vals, gidx = [], []
    for j in range(_J):
        m, wdep = tournament(t)
        vals.append(m)
        gidx.append(wdep * csz + si)
        if j < _J - 1:
            t = [jnp.where(wdep == w, jnp.inf, t[w]) for w in range(_W)]
    cand_v = jnp.concatenate(vals, axis=0)
    cand_i = jnp.concatenate(gidx, axis=0)

    rows = []
    m = None
    for _ in range(k):
        m = jnp.min(cand_v, axis=0, keepdims=True)
        idx = jnp.min(jnp.where(cand_v == m, cand_i, _BIG), axis=0,
                      keepdims=True)
        rows.append(idx)
        cand_v = jnp.where(cand_i == idx, jnp.inf, cand_v)
    out_ref[:, :] = jnp.concatenate(rows, axis=0)

    bad = jnp.any(vals[-1] <= m)

    @pl.when(bad)
    def _fallback():
        tt = jnp.concatenate([dist_slice(w) for w in range(_W)], axis=0)
        iota = jax.lax.broadcasted_iota(jnp.int32, tt.shape, 0)
        frows = []
        for _ in range(k):
            fm = jnp.min(tt, axis=0, keepdims=True)
            fidx = jnp.min(jnp.where(tt == fm, iota, _BIG), axis=0,
                           keepdims=True)
            frows.append(fidx)
            tt = jnp.where(iota == fidx, jnp.inf, tt)
        out_ref[:, :] = jnp.concatenate(frows, axis=0)


@jax.jit
def kernel(p, active_nodes):
    n = p.shape[0]
    pcols = jnp.zeros((8, n), dtype=p.dtype).at[:3, :].set(p.T)
    idxs_t = pl.pallas_call(
        functools.partial(_knn_block_kernel, n=n, k=_K),
        grid=(n // _BLOCK_R,),
        in_specs=[
            pl.BlockSpec((n, 3), lambda i: (0, 0)),
            pl.BlockSpec((8, _BLOCK_R), lambda i: (0, i)),
        ],
        out_specs=pl.BlockSpec((_K, _BLOCK_R), lambda i: (0, i)),
        out_shape=jax.ShapeDtypeStruct((_K, n), jnp.int32),
        compiler_params=pltpu.CompilerParams(
            dimension_semantics=("arbitrary",),
        ),
    )(p, pcols)
    idxs = idxs_t.T
    row = jnp.broadcast_to(jnp.arange(n, dtype=idxs.dtype)[:, None], (n, _K))
    s = jnp.where(active_nodes[:, None], idxs, n - 1)
    r = jnp.where(active_nodes[:, None], row, n - 1)
    return s.reshape(-1), r.reshape(-1)
